# baseline (speedup 1.0000x reference)
import jax
import jax.numpy as jnp
from jax import lax
from jax.experimental import pallas as pl
from jax.experimental.pallas import tpu as pltpu
from jax.experimental.pallas import tpu_sc as plsc

N = 10000
E = 320000
NR = 8
D = 128
OUT = 128
EPB = E // NR

NC = 2
NS = 16
L = 16
NW = NC * NS
EPT = E // NW
K = 80
NB = EPT // K
GROUPS = K // L
CH = OUT // L
STRIPE = 632
LAST_STRIPE = N - (NS - 1) * STRIPE
RB = 3
NB_MAIN = (NB - 2) // RB * RB

BN = 1000


def _ymat_body(nf_ref, ef_ref, wr_ref, y_ref):
    h = nf_ref[...]
    r = ef_ref[0]
    r_re = r[:, : D // 2]
    r_im = r[:, D // 2 :]
    a = jnp.concatenate([r_re, r_re], axis=1)
    b = jnp.concatenate([-r_im, r_im], axis=1)
    h_swap = jnp.concatenate([h[:, D // 2 :], h[:, : D // 2]], axis=1)
    z = h * a + h_swap * b
    y_ref[0] = jnp.dot(z, wr_ref[0], preferred_element_type=jnp.float32)


def _stage1(node_feat, edge_feat, w_r):
    return pl.pallas_call(
        _ymat_body,
        grid=(N // BN, NR),
        in_specs=[
            pl.BlockSpec((BN, D), lambda i, t: (i, 0)),
            pl.BlockSpec((1, 1, D), lambda i, t: (t, 0, 0)),
            pl.BlockSpec((1, D, OUT), lambda i, t: (t, 0, 0)),
        ],
        out_specs=pl.BlockSpec((1, BN, OUT), lambda i, t: (t, i, 0)),
        out_shape=jax.ShapeDtypeStruct((NR, N, OUT), jnp.float32),
    )(node_feat, edge_feat.reshape(NR, 1, D), w_r)


def _edge_body(src_hbm, dst_hbm, norm_hbm, y_hbm, zeros_hbm, out_hbm,
               src_big,
               rows0, rows1, rows2,
               dstv0, dstv1, dstv2,
               normv0, normv1, normv2,
               acc_sh,
               gs0, gs1, gs2, ds0, ds1, ds2,
               ns0, ns1, ns2, ss0, ss1, ss2, zsem):
    rows = (rows0, rows1, rows2)
    dsts = (dstv0, dstv1, dstv2)
    norms = (normv0, normv1, normv2)
    gsems = (gs0, gs1, gs2)
    dsems = (ds0, ds1, ds2)
    nsems = (ns0, ns1, ns2)
    ssems = (ss0, ss1, ss2)

    c = lax.axis_index("c")
    s = lax.axis_index("s")
    wid = s * NC + c
    base = wid * EPT
    t = wid // (EPB // EPT)
    t_off = jnp.full((L,), t * N, jnp.int32)
    r0 = s * STRIPE

    @pl.when(s < NS - 1)
    def _():
        pltpu.async_copy(zeros_hbm.at[pl.ds(r0, STRIPE)],
                         acc_sh.at[pl.ds(r0, STRIPE)], zsem)

    @pl.when(s == NS - 1)
    def _():
        pltpu.async_copy(zeros_hbm.at[pl.ds((NS - 1) * STRIPE, LAST_STRIPE)],
                         acc_sh.at[pl.ds((NS - 1) * STRIPE, LAST_STRIPE)], zsem)

    pltpu.sync_copy(src_hbm.at[pl.ds(base, EPT)], src_big)

    def adj(i, carry):
        sl = pl.ds(i * L, L)
        src_big[sl] = src_big[sl] + t_off
        return carry

    lax.fori_loop(0, EPT // L, adj, 0, unroll=8)

    @pl.when(s < NS - 1)
    def _():
        pltpu.make_async_copy(zeros_hbm.at[pl.ds(r0, STRIPE)],
                              acc_sh.at[pl.ds(r0, STRIPE)], zsem).wait()

    @pl.when(s == NS - 1)
    def _():
        pltpu.make_async_copy(
            zeros_hbm.at[pl.ds((NS - 1) * STRIPE, LAST_STRIPE)],
            acc_sh.at[pl.ds((NS - 1) * STRIPE, LAST_STRIPE)], zsem).wait()

    plsc.subcore_barrier()

    lanes = lax.iota(jnp.int32, L)

    def fire(bf, x):
        off = base + bf * K
        pltpu.async_copy(dst_hbm.at[pl.ds(off, K)], dsts[x], dsems[x])
        pltpu.async_copy(norm_hbm.at[pl.ds(off, K)], norms[x], nsems[x])
        pltpu.async_copy(y_hbm.at[src_big.at[pl.ds(bf * K, K)]], rows[x],
                         gsems[x])

    def step(b, u):
        xf = (u + 1) % RB

        @pl.when(b >= 2)
        def _():
            pltpu.make_async_copy(rows[xf], acc_sh.at[dsts[xf]],
                                  ssems[xf]).wait()

        @pl.when(b + 1 < NB)
        def _():
            fire(b + 1, xf)

        pltpu.make_async_copy(y_hbm.at[src_big.at[pl.ds(0, K)]], rows[u],
                              gsems[u]).wait()
        pltpu.make_async_copy(norm_hbm.at[pl.ds(base, K)], norms[u],
                              nsems[u]).wait()

        def grp(g, carry2):
            for r in range(L):
                i = g * L + r
                nb = plsc.load_gather(norms[u], [jnp.full((L,), i, jnp.int32)])
                for c2 in range(CH):
                    csl = pl.ds(c2 * L, L)
                    rows[u][i, csl] = rows[u][i, csl] * nb
            return carry2

        lax.fori_loop(0, GROUPS, grp, 0)

        pltpu.make_async_copy(dst_hbm.at[pl.ds(base, K)], dsts[u],
                              dsems[u]).wait()
        pltpu.async_copy(rows[u], acc_sh.at[dsts[u]], ssems[u], add=True)

    fire(0, 0)

    def body(j, carry):
        for u in range(RB):
            step(RB * j + u, u)
        return carry

    lax.fori_loop(0, NB_MAIN // RB, body, 0)

    for b in range(NB_MAIN, NB):
        step(b, b % RB)

    for b in (NB - 2, NB - 1):
        x = b % RB
        pltpu.make_async_copy(rows[x], acc_sh.at[dsts[x]], ssems[x]).wait()

    plsc.subcore_barrier()

    @pl.when(s < NS - 1)
    def _():
        pltpu.sync_copy(acc_sh.at[pl.ds(r0, STRIPE)],
                        out_hbm.at[c, pl.ds(r0, STRIPE)])

    @pl.when(s == NS - 1)
    def _():
        pltpu.sync_copy(acc_sh.at[pl.ds((NS - 1) * STRIPE, LAST_STRIPE)],
                        out_hbm.at[c, pl.ds((NS - 1) * STRIPE, LAST_STRIPE)])


def _stage2(src, dst, norm, y_flat, zeros):
    mesh = plsc.VectorSubcoreMesh(core_axis_name="c", subcore_axis_name="s",
                                  num_cores=NC, num_subcores=NS)
    fn = pl.kernel(
        _edge_body,
        out_type=jax.ShapeDtypeStruct((NC, N, OUT), jnp.float32),
        mesh=mesh,
        scratch_types=(
            [pltpu.VMEM((EPT,), jnp.int32)]
            + [pltpu.VMEM((K, OUT), jnp.float32)] * RB
            + [pltpu.VMEM((K,), jnp.int32)] * RB
            + [pltpu.VMEM((K,), jnp.float32)] * RB
            + [pltpu.VMEM_SHARED((N, OUT), jnp.float32)]
            + [pltpu.SemaphoreType.DMA] * (4 * RB + 1)
        ),
        compiler_params=pltpu.CompilerParams(needs_layout_passes=False),
    )
    return fn(src, dst, norm, y_flat, zeros)


def _bn_body(hp_ref, ef_ref, wrel_ref, g_ref, b_ref, o1_ref, o2_ref):
    h = hp_ref[0] + hp_ref[1]
    mean = jnp.mean(h, axis=0, keepdims=True)
    var = jnp.mean((h - mean) ** 2, axis=0, keepdims=True)
    x = (h - mean) * lax.rsqrt(var + 1e-5) * g_ref[...] + b_ref[...]
    o1_ref[...] = jnp.tanh(x)
    o2_ref[...] = jnp.dot(ef_ref[...], wrel_ref[...],
                          preferred_element_type=jnp.float32)


def _stage3(hp, edge_feat, w_rel, gamma2, beta2):
    return pl.pallas_call(
        _bn_body,
        out_shape=(
            jax.ShapeDtypeStruct((N, OUT), jnp.float32),
            jax.ShapeDtypeStruct((NR, OUT), jnp.float32),
        ),
    )(hp, edge_feat, w_rel, gamma2, beta2)


def kernel(node_feat, edge_feat, etype_norm, W_R, W_rel, gamma, beta, edge_index):
    y = _stage1(node_feat, edge_feat, W_R)
    y_flat = y.reshape(NR * N, OUT)
    zeros = jnp.zeros((N, OUT), jnp.float32)
    hp = _stage2(edge_index[0], edge_index[1], etype_norm, y_flat, zeros)
    out1, out2 = _stage3(hp, edge_feat, W_rel,
                         gamma.reshape(1, OUT), beta.reshape(1, OUT))
    return (out1, out2)


# speedup vs baseline: 1.0768x; 1.0768x over previous
import jax
import jax.numpy as jnp
from jax import lax
from jax.experimental import pallas as pl
from jax.experimental.pallas import tpu as pltpu
from jax.experimental.pallas import tpu_sc as plsc

N = 10000
E = 320000
NR = 8
D = 128
OUT = 128
EPB = E // NR

NC = 2
NS = 16
L = 16
NW = NC * NS
EPT = E // NW
K = 80
NB = EPT // K
GROUPS = K // L
CH = OUT // L
STRIPE = 632
LAST_STRIPE = N - (NS - 1) * STRIPE
RB = 3
NB_MAIN = (NB - 2) // RB * RB

BN = 1000


def _ymat_body(nf_ref, ef_ref, wr_ref, y_ref):
    h = nf_ref[...]
    r = ef_ref[0]
    r_re = r[:, : D // 2]
    r_im = r[:, D // 2 :]
    a = jnp.concatenate([r_re, r_re], axis=1)
    b = jnp.concatenate([-r_im, r_im], axis=1)
    h_swap = jnp.concatenate([h[:, D // 2 :], h[:, : D // 2]], axis=1)
    z = h * a + h_swap * b
    y_ref[0] = jnp.dot(z, wr_ref[0], preferred_element_type=jnp.float32)


def _stage1(node_feat, edge_feat, w_r):
    return pl.pallas_call(
        _ymat_body,
        grid=(N // BN, NR),
        in_specs=[
            pl.BlockSpec((BN, D), lambda i, t: (i, 0)),
            pl.BlockSpec((1, 1, D), lambda i, t: (t, 0, 0)),
            pl.BlockSpec((1, D, OUT), lambda i, t: (t, 0, 0)),
        ],
        out_specs=pl.BlockSpec((1, BN, OUT), lambda i, t: (t, i, 0)),
        out_shape=jax.ShapeDtypeStruct((NR, N, OUT), jnp.float32),
    )(node_feat, edge_feat.reshape(NR, 1, D), w_r)


def _edge_body(src_hbm, dst_hbm, norm_hbm, y_hbm, zeros_hbm, out_hbm,
               src_big,
               rows0, rows1, rows2,
               dstv0, dstv1, dstv2,
               normv0, normv1, normv2,
               acc_sh,
               gs0, gs1, gs2, ds0, ds1, ds2,
               ns0, ns1, ns2, ss0, ss1, ss2, zsem):
    rows = (rows0, rows1, rows2)
    dsts = (dstv0, dstv1, dstv2)
    norms = (normv0, normv1, normv2)
    gsems = (gs0, gs1, gs2)
    dsems = (ds0, ds1, ds2)
    nsems = (ns0, ns1, ns2)
    ssems = (ss0, ss1, ss2)

    c = lax.axis_index("c")
    s = lax.axis_index("s")
    wid = s * NC + c
    base = wid * EPT
    t = wid // (EPB // EPT)
    t_off = jnp.full((L,), t * N, jnp.int32)
    r0 = s * STRIPE

    @pl.when(s < NS - 1)
    def _():
        pltpu.async_copy(zeros_hbm.at[pl.ds(r0, STRIPE)],
                         acc_sh.at[pl.ds(r0, STRIPE)], zsem)

    @pl.when(s == NS - 1)
    def _():
        pltpu.async_copy(zeros_hbm.at[pl.ds((NS - 1) * STRIPE, LAST_STRIPE)],
                         acc_sh.at[pl.ds((NS - 1) * STRIPE, LAST_STRIPE)], zsem)

    pltpu.sync_copy(src_hbm.at[pl.ds(base, EPT)], src_big)

    def adj(i, carry):
        sl = pl.ds(i * L, L)
        src_big[sl] = src_big[sl] + t_off
        return carry

    lax.fori_loop(0, EPT // L, adj, 0, unroll=8)

    @pl.when(s < NS - 1)
    def _():
        pltpu.make_async_copy(zeros_hbm.at[pl.ds(r0, STRIPE)],
                              acc_sh.at[pl.ds(r0, STRIPE)], zsem).wait()

    @pl.when(s == NS - 1)
    def _():
        pltpu.make_async_copy(
            zeros_hbm.at[pl.ds((NS - 1) * STRIPE, LAST_STRIPE)],
            acc_sh.at[pl.ds((NS - 1) * STRIPE, LAST_STRIPE)], zsem).wait()

    plsc.subcore_barrier()

    lanes = lax.iota(jnp.int32, L)

    def fire(bf, x):
        off = base + bf * K
        pltpu.async_copy(dst_hbm.at[pl.ds(off, K)], dsts[x], dsems[x])
        pltpu.async_copy(norm_hbm.at[pl.ds(off, K)], norms[x], nsems[x])
        pltpu.async_copy(y_hbm.at[src_big.at[pl.ds(bf * K, K)]], rows[x],
                         gsems[x])

    def step(b, u):
        xf = (u + 1) % RB

        @pl.when(b >= 2)
        def _():
            pltpu.make_async_copy(rows[xf], acc_sh.at[dsts[xf]],
                                  ssems[xf]).wait()

        @pl.when(b + 1 < NB)
        def _():
            fire(b + 1, xf)

        pltpu.make_async_copy(y_hbm.at[src_big.at[pl.ds(0, K)]], rows[u],
                              gsems[u]).wait()
        pltpu.make_async_copy(norm_hbm.at[pl.ds(base, K)], norms[u],
                              nsems[u]).wait()

        def grp(g, carry2):
            n16 = norms[u][pl.ds(g * L, L)]
            for r in range(L):
                nb = jnp.sum(jnp.where(lanes == r, n16, 0.0))
                i = g * L + r
                for c2 in range(CH):
                    csl = pl.ds(c2 * L, L)
                    rows[u][i, csl] = rows[u][i, csl] * nb
            return carry2

        lax.fori_loop(0, GROUPS, grp, 0)

        pltpu.make_async_copy(dst_hbm.at[pl.ds(base, K)], dsts[u],
                              dsems[u]).wait()
        pltpu.async_copy(rows[u], acc_sh.at[dsts[u]], ssems[u], add=True)

    fire(0, 0)

    def body(j, carry):
        for u in range(RB):
            step(RB * j + u, u)
        return carry

    lax.fori_loop(0, NB_MAIN // RB, body, 0)

    for b in range(NB_MAIN, NB):
        step(b, b % RB)

    for b in (NB - 2, NB - 1):
        x = b % RB
        pltpu.make_async_copy(rows[x], acc_sh.at[dsts[x]], ssems[x]).wait()

    plsc.subcore_barrier()

    @pl.when(s < NS - 1)
    def _():
        pltpu.sync_copy(acc_sh.at[pl.ds(r0, STRIPE)],
                        out_hbm.at[c, pl.ds(r0, STRIPE)])

    @pl.when(s == NS - 1)
    def _():
        pltpu.sync_copy(acc_sh.at[pl.ds((NS - 1) * STRIPE, LAST_STRIPE)],
                        out_hbm.at[c, pl.ds((NS - 1) * STRIPE, LAST_STRIPE)])


def _stage2(src, dst, norm, y_flat, zeros):
    mesh = plsc.VectorSubcoreMesh(core_axis_name="c", subcore_axis_name="s",
                                  num_cores=NC, num_subcores=NS)
    fn = pl.kernel(
        _edge_body,
        out_type=jax.ShapeDtypeStruct((NC, N, OUT), jnp.float32),
        mesh=mesh,
        scratch_types=(
            [pltpu.VMEM((EPT,), jnp.int32)]
            + [pltpu.VMEM((K, OUT), jnp.float32)] * RB
            + [pltpu.VMEM((K,), jnp.int32)] * RB
            + [pltpu.VMEM((K,), jnp.float32)] * RB
            + [pltpu.VMEM_SHARED((N, OUT), jnp.float32)]
            + [pltpu.SemaphoreType.DMA] * (4 * RB + 1)
        ),
        compiler_params=pltpu.CompilerParams(needs_layout_passes=False),
    )
    return fn(src, dst, norm, y_flat, zeros)


def _bn_body(hp_ref, ef_ref, wrel_ref, g_ref, b_ref, o1_ref, o2_ref):
    h = hp_ref[0] + hp_ref[1]
    mean = jnp.mean(h, axis=0, keepdims=True)
    var = jnp.mean((h - mean) ** 2, axis=0, keepdims=True)
    x = (h - mean) * lax.rsqrt(var + 1e-5) * g_ref[...] + b_ref[...]
    o1_ref[...] = jnp.tanh(x)
    o2_ref[...] = jnp.dot(ef_ref[...], wrel_ref[...],
                          preferred_element_type=jnp.float32)


def _stage3(hp, edge_feat, w_rel, gamma2, beta2):
    return pl.pallas_call(
        _bn_body,
        out_shape=(
            jax.ShapeDtypeStruct((N, OUT), jnp.float32),
            jax.ShapeDtypeStruct((NR, OUT), jnp.float32),
        ),
    )(hp, edge_feat, w_rel, gamma2, beta2)


def kernel(node_feat, edge_feat, etype_norm, W_R, W_rel, gamma, beta, edge_index):
    y = _stage1(node_feat, edge_feat, W_R)
    y_flat = y.reshape(NR * N, OUT)
    zeros = jnp.zeros((N, OUT), jnp.float32)
    hp = _stage2(edge_index[0], edge_index[1], etype_norm, y_flat, zeros)
    out1, out2 = _stage3(hp, edge_feat, W_rel,
                         gamma.reshape(1, OUT), beta.reshape(1, OUT))
    return (out1, out2)


# speedup vs baseline: 1.0963x; 1.0181x over previous
import jax
import jax.numpy as jnp
from jax import lax
from jax.experimental import pallas as pl
from jax.experimental.pallas import tpu as pltpu
from jax.experimental.pallas import tpu_sc as plsc

N = 10000
E = 320000
NR = 8
D = 128
OUT = 128
EPB = E // NR

NC = 2
NS = 16
L = 16
NW = NC * NS
EPT = E // NW
K = 80
NB = EPT // K
GROUPS = K // L
CH = OUT // L
STRIPE = 632
LAST_STRIPE = N - (NS - 1) * STRIPE
RB = 3
NB_MAIN = (NB - 2) // RB * RB

BN = 1000


def _ymat_body(nf_ref, ef_ref, wr_ref, y_ref):
    h = nf_ref[...]
    r = ef_ref[0]
    r_re = r[:, : D // 2]
    r_im = r[:, D // 2 :]
    a = jnp.concatenate([r_re, r_re], axis=1)
    b = jnp.concatenate([-r_im, r_im], axis=1)
    h_swap = jnp.concatenate([h[:, D // 2 :], h[:, : D // 2]], axis=1)
    z = h * a + h_swap * b
    y_ref[0] = jnp.dot(z, wr_ref[0], preferred_element_type=jnp.float32)


def _stage1(node_feat, edge_feat, w_r):
    return pl.pallas_call(
        _ymat_body,
        grid=(N // BN, NR),
        in_specs=[
            pl.BlockSpec((BN, D), lambda i, t: (i, 0)),
            pl.BlockSpec((1, 1, D), lambda i, t: (t, 0, 0)),
            pl.BlockSpec((1, D, OUT), lambda i, t: (t, 0, 0)),
        ],
        out_specs=pl.BlockSpec((1, BN, OUT), lambda i, t: (t, i, 0)),
        out_shape=jax.ShapeDtypeStruct((NR, N, OUT), jnp.float32),
    )(node_feat, edge_feat.reshape(NR, 1, D), w_r)


def _edge_body(src_hbm, dst_hbm, norm_hbm, y_hbm, out_hbm,
               src_big,
               rows0, rows1, rows2,
               dstv0, dstv1, dstv2,
               normv0, normv1, normv2,
               acc_sh,
               gs0, gs1, gs2, ds0, ds1, ds2,
               ns0, ns1, ns2, ss0, ss1, ss2):
    rows = (rows0, rows1, rows2)
    dsts = (dstv0, dstv1, dstv2)
    norms = (normv0, normv1, normv2)
    gsems = (gs0, gs1, gs2)
    dsems = (ds0, ds1, ds2)
    nsems = (ns0, ns1, ns2)
    ssems = (ss0, ss1, ss2)

    c = lax.axis_index("c")
    s = lax.axis_index("s")
    wid = s * NC + c
    base = wid * EPT
    t = wid // (EPB // EPT)
    t_off = jnp.full((L,), t * N, jnp.int32)
    r0 = s * STRIPE

    pltpu.sync_copy(src_hbm.at[pl.ds(base, EPT)], src_big)

    def adj(i, carry):
        sl = pl.ds(i * L, L)
        src_big[sl] = src_big[sl] + t_off
        return carry

    lax.fori_loop(0, EPT // L, adj, 0, unroll=8)

    zero16 = jnp.zeros((L,), jnp.float32)

    def zrow(i, carry):
        for c2 in range(CH):
            rows0[i, pl.ds(c2 * L, L)] = zero16
        return carry

    lax.fori_loop(0, K, zrow, 0)

    @pl.when(s < NS - 1)
    def _():
        for zj in range(STRIPE // K):
            pltpu.sync_copy(rows0.at[pl.ds(0, K)],
                            acc_sh.at[pl.ds(r0 + zj * K, K)])
        pltpu.sync_copy(rows0.at[pl.ds(0, STRIPE % K)],
                        acc_sh.at[pl.ds(r0 + (STRIPE // K) * K, STRIPE % K)])

    @pl.when(s == NS - 1)
    def _():
        lr0 = (NS - 1) * STRIPE
        for zj in range(LAST_STRIPE // K):
            pltpu.sync_copy(rows0.at[pl.ds(0, K)],
                            acc_sh.at[pl.ds(lr0 + zj * K, K)])
        pltpu.sync_copy(rows0.at[pl.ds(0, LAST_STRIPE % K)],
                        acc_sh.at[pl.ds(lr0 + (LAST_STRIPE // K) * K,
                                        LAST_STRIPE % K)])

    plsc.subcore_barrier()

    lanes = lax.iota(jnp.int32, L)

    def fire(bf, x):
        off = base + bf * K
        pltpu.async_copy(dst_hbm.at[pl.ds(off, K)], dsts[x], dsems[x])
        pltpu.async_copy(norm_hbm.at[pl.ds(off, K)], norms[x], nsems[x])
        pltpu.async_copy(y_hbm.at[src_big.at[pl.ds(bf * K, K)]], rows[x],
                         gsems[x])

    def step(b, u):
        xf = (u + 1) % RB

        @pl.when(b >= 2)
        def _():
            pltpu.make_async_copy(rows[xf], acc_sh.at[dsts[xf]],
                                  ssems[xf]).wait()

        @pl.when(b + 1 < NB)
        def _():
            fire(b + 1, xf)

        pltpu.make_async_copy(y_hbm.at[src_big.at[pl.ds(0, K)]], rows[u],
                              gsems[u]).wait()
        pltpu.make_async_copy(norm_hbm.at[pl.ds(base, K)], norms[u],
                              nsems[u]).wait()

        def grp(g, carry2):
            n16 = norms[u][pl.ds(g * L, L)]
            for r in range(L):
                nb = jnp.sum(jnp.where(lanes == r, n16, 0.0))
                i = g * L + r
                for c2 in range(CH):
                    csl = pl.ds(c2 * L, L)
                    rows[u][i, csl] = rows[u][i, csl] * nb
            return carry2

        lax.fori_loop(0, GROUPS, grp, 0)

        pltpu.make_async_copy(dst_hbm.at[pl.ds(base, K)], dsts[u],
                              dsems[u]).wait()
        pltpu.async_copy(rows[u], acc_sh.at[dsts[u]], ssems[u], add=True)

    fire(0, 0)

    def body(j, carry):
        for u in range(RB):
            step(RB * j + u, u)
        return carry

    lax.fori_loop(0, NB_MAIN // RB, body, 0)

    for b in range(NB_MAIN, NB):
        step(b, b % RB)

    for b in (NB - 2, NB - 1):
        x = b % RB
        pltpu.make_async_copy(rows[x], acc_sh.at[dsts[x]], ssems[x]).wait()

    plsc.subcore_barrier()

    @pl.when(s < NS - 1)
    def _():
        pltpu.sync_copy(acc_sh.at[pl.ds(r0, STRIPE)],
                        out_hbm.at[c, pl.ds(r0, STRIPE)])

    @pl.when(s == NS - 1)
    def _():
        pltpu.sync_copy(acc_sh.at[pl.ds((NS - 1) * STRIPE, LAST_STRIPE)],
                        out_hbm.at[c, pl.ds((NS - 1) * STRIPE, LAST_STRIPE)])


def _stage2(src, dst, norm, y_flat):
    mesh = plsc.VectorSubcoreMesh(core_axis_name="c", subcore_axis_name="s",
                                  num_cores=NC, num_subcores=NS)
    fn = pl.kernel(
        _edge_body,
        out_type=jax.ShapeDtypeStruct((NC, N, OUT), jnp.float32),
        mesh=mesh,
        scratch_types=(
            [pltpu.VMEM((EPT,), jnp.int32)]
            + [pltpu.VMEM((K, OUT), jnp.float32)] * RB
            + [pltpu.VMEM((K,), jnp.int32)] * RB
            + [pltpu.VMEM((K,), jnp.float32)] * RB
            + [pltpu.VMEM_SHARED((N, OUT), jnp.float32)]
            + [pltpu.SemaphoreType.DMA] * (4 * RB)
        ),
        compiler_params=pltpu.CompilerParams(needs_layout_passes=False),
    )
    return fn(src, dst, norm, y_flat)


def _bn_body(hp_ref, ef_ref, wrel_ref, g_ref, b_ref, o1_ref, o2_ref):
    h = hp_ref[0] + hp_ref[1]
    mean = jnp.mean(h, axis=0, keepdims=True)
    var = jnp.mean((h - mean) ** 2, axis=0, keepdims=True)
    x = (h - mean) * lax.rsqrt(var + 1e-5) * g_ref[...] + b_ref[...]
    o1_ref[...] = jnp.tanh(x)
    o2_ref[...] = jnp.dot(ef_ref[...], wrel_ref[...],
                          preferred_element_type=jnp.float32)


def _stage3(hp, edge_feat, w_rel, gamma2, beta2):
    return pl.pallas_call(
        _bn_body,
        out_shape=(
            jax.ShapeDtypeStruct((N, OUT), jnp.float32),
            jax.ShapeDtypeStruct((NR, OUT), jnp.float32),
        ),
    )(hp, edge_feat, w_rel, gamma2, beta2)


def kernel(node_feat, edge_feat, etype_norm, W_R, W_rel, gamma, beta, edge_index):
    y = _stage1(node_feat, edge_feat, W_R)
    y_flat = y.reshape(NR * N, OUT)
    hp = _stage2(edge_index[0], edge_index[1], etype_norm, y_flat)
    out1, out2 = _stage3(hp, edge_feat, W_rel,
                         gamma.reshape(1, OUT), beta.reshape(1, OUT))
    return (out1, out2)
